# baseline (speedup 1.0000x reference)
import jax
import jax.numpy as jnp
from jax.experimental import pallas as pl
from jax.experimental.pallas import tpu as pltpu

_BM = 1024
_NBUF = 4


def _copy(x_hbm, xbuf, sems, step, slot):
    base = step * 2 * _BM + _BM
    return pltpu.make_async_copy(
        x_hbm.at[pl.ds(base, _BM), :], xbuf.at[slot], sems.at[slot])


def _gate_matmul(x_top_ref, x_hbm, wgt_ref, o_ref, xbuf, sems):
    i = pl.program_id(0)
    nsteps = pl.num_programs(0)

    @pl.when(i == 0)
    def _warmup():
        for b in range(_NBUF):
            _copy(x_hbm, xbuf, sems, b, b).start()

    o_ref[:_BM, :] = jnp.dot(x_top_ref[...], wgt_ref[...],
                             preferred_element_type=jnp.float32)

    slot = jax.lax.rem(i, _NBUF)
    _copy(x_hbm, xbuf, sems, i, slot).wait()
    o_ref[_BM:, :] = jnp.dot(xbuf[slot], wgt_ref[...],
                             preferred_element_type=jnp.float32)

    nxt = i + _NBUF

    @pl.when(nxt < nsteps)
    def _prefetch():
        _copy(x_hbm, xbuf, sems, nxt, slot).start()


def kernel(x, wg):
    m, k = x.shape
    e = wg.shape[0]
    wgt = wg.T
    nsteps = m // (2 * _BM)
    return pl.pallas_call(
        _gate_matmul,
        grid=(nsteps,),
        in_specs=[
            pl.BlockSpec((_BM, k), lambda i: (2 * i, 0)),
            pl.BlockSpec(memory_space=pl.ANY),
            pl.BlockSpec((k, e), lambda i: (0, 0)),
        ],
        out_specs=pl.BlockSpec((2 * _BM, e), lambda i: (i, 0)),
        out_shape=jax.ShapeDtypeStruct((m, e), jnp.float32),
        scratch_shapes=[
            pltpu.VMEM((_NBUF, _BM, k), jnp.float32),
            pltpu.SemaphoreType.DMA((_NBUF,)),
        ],
    )(x, x, wgt)


# speedup vs baseline: 1.0757x; 1.0757x over previous
import jax
import jax.numpy as jnp
from jax.experimental import pallas as pl
from jax.experimental.pallas import tpu as pltpu

_BM = 512
_NBUF = 16


def _copy(x_hbm, xbuf, sems, block, slot):
    return pltpu.make_async_copy(
        x_hbm.at[pl.ds(block * _BM, _BM), :], xbuf.at[slot], sems.at[slot])


def _gate_matmul(x_hbm, wgt_ref, o_ref, xbuf, sems):
    i = pl.program_id(0)
    nsteps = pl.num_programs(0)

    @pl.when(i == 0)
    def _warmup():
        for b in range(_NBUF):
            _copy(x_hbm, xbuf, sems, b, b).start()

    slot = jax.lax.rem(i, _NBUF)
    _copy(x_hbm, xbuf, sems, i, slot).wait()
    o_ref[...] = jnp.dot(xbuf[slot], wgt_ref[...],
                         preferred_element_type=jnp.float32)

    nxt = i + _NBUF

    @pl.when(nxt < nsteps)
    def _prefetch():
        _copy(x_hbm, xbuf, sems, nxt, slot).start()


def kernel(x, wg):
    m, k = x.shape
    e = wg.shape[0]
    wgt = wg.T
    return pl.pallas_call(
        _gate_matmul,
        grid=(m // _BM,),
        in_specs=[
            pl.BlockSpec(memory_space=pl.ANY),
            pl.BlockSpec((k, e), lambda i: (0, 0)),
        ],
        out_specs=pl.BlockSpec((_BM, e), lambda i: (i, 0)),
        out_shape=jax.ShapeDtypeStruct((m, e), jnp.float32),
        scratch_shapes=[
            pltpu.VMEM((_NBUF, _BM, k), jnp.float32),
            pltpu.SemaphoreType.DMA((_NBUF,)),
        ],
    )(x, wgt)


# speedup vs baseline: 1.5281x; 1.4205x over previous
import jax
import jax.numpy as jnp
from jax.experimental import pallas as pl
from jax.experimental.pallas import tpu as pltpu

_BM = 1024
_NBUF = 8


def _copy(x_hbm, xbuf, sems, block, slot):
    return pltpu.make_async_copy(
        x_hbm.at[pl.ds(block * _BM, _BM), :], xbuf.at[slot], sems.at[slot])


def _gate_matmul(x_hbm, wgt_ref, o_ref, xbuf, sems):
    i = pl.program_id(0)
    nsteps = pl.num_programs(0)

    @pl.when(i == 0)
    def _warmup():
        for b in range(_NBUF):
            _copy(x_hbm, xbuf, sems, b, b).start()

    slot = jax.lax.rem(i, _NBUF)
    _copy(x_hbm, xbuf, sems, i, slot).wait()
    o_ref[...] = jnp.dot(xbuf[slot], wgt_ref[...],
                         preferred_element_type=jnp.float32).T

    nxt = i + _NBUF

    @pl.when(nxt < nsteps)
    def _prefetch():
        _copy(x_hbm, xbuf, sems, nxt, slot).start()


def kernel(x, wg):
    m, k = x.shape
    e = wg.shape[0]
    wgt = wg.T
    out_t = pl.pallas_call(
        _gate_matmul,
        grid=(m // _BM,),
        in_specs=[
            pl.BlockSpec(memory_space=pl.ANY),
            pl.BlockSpec((k, e), lambda i: (0, 0)),
        ],
        out_specs=pl.BlockSpec((e, _BM), lambda i: (0, i)),
        out_shape=jax.ShapeDtypeStruct((e, m), jnp.float32),
        scratch_shapes=[
            pltpu.VMEM((_NBUF, _BM, k), jnp.float32),
            pltpu.SemaphoreType.DMA((_NBUF,)),
        ],
    )(x, wgt)
    return out_t.T
